# baseline (speedup 1.0000x reference)
import functools

import jax
import jax.numpy as jnp
from jax import lax
from jax.experimental import pallas as pl
from jax.experimental.pallas import tpu as pltpu
from jax.experimental.pallas import tpu_sc as plsc

DIM = 64
BATCH = 16384
NUM_CORES = 2
NUM_SUBCORES = 16
LANES = 16
NW = NUM_CORES * NUM_SUBCORES
BPW = BATCH // NW
CHUNK = 128
NCHUNK = BPW // CHUNK
GROUPS = BPW // LANES


def _fmc_body(prev_hbm, pos_hbm, neg_hbm, win_hbm, wout_hbm, out_hbm,
              pidx, aidx, bidx, hrows, prows, nrows, outv, sem):
    wid = lax.axis_index("s") * NUM_CORES + lax.axis_index("c")
    base = wid * BPW

    pltpu.sync_copy(prev_hbm.at[pl.ds(base, BPW)], pidx)
    pltpu.sync_copy(pos_hbm.at[pl.ds(base, BPW)], aidx)
    pltpu.sync_copy(neg_hbm.at[pl.ds(base, BPW)], bidx)

    copies = []
    for c in range(NCHUNK):
        sl = pl.ds(c * CHUNK, CHUNK)
        copies.append(pltpu.make_async_copy(win_hbm.at[pidx.at[sl]],
                                            hrows.at[sl], sem))
        copies.append(pltpu.make_async_copy(wout_hbm.at[aidx.at[sl]],
                                            prows.at[sl], sem))
        copies.append(pltpu.make_async_copy(wout_hbm.at[bidx.at[sl]],
                                            nrows.at[sl], sem))
    for cp in copies:
        cp.start()
    for cp in copies:
        cp.wait()

    iota = lax.iota(jnp.int32, LANES)

    def group(g, _):
        rows = g * LANES + iota
        acc = jnp.zeros((LANES,), jnp.float32)
        for d in range(DIM):
            col = jnp.full((LANES,), d, jnp.int32)
            hv = plsc.load_gather(hrows, [rows, col])
            pv = plsc.load_gather(prows, [rows, col])
            nv = plsc.load_gather(nrows, [rows, col])
            acc = acc + hv * (pv - nv)
        outv[pl.ds(g * LANES, LANES)] = acc
        return 0

    lax.fori_loop(0, GROUPS, group, 0)

    pltpu.sync_copy(outv, out_hbm.at[pl.ds(base, BPW)])


@jax.jit
def _fmc(prev, pos, neg, item_in_weight, item_out_weight):
    mesh = plsc.VectorSubcoreMesh(core_axis_name="c", subcore_axis_name="s")
    run = functools.partial(
        pl.kernel,
        out_type=jax.ShapeDtypeStruct((BATCH,), jnp.float32),
        mesh=mesh,
        compiler_params=pltpu.CompilerParams(needs_layout_passes=False,
                                             use_tc_tiling_on_sc=False),
        scratch_types=[
            pltpu.VMEM((BPW,), jnp.int32),
            pltpu.VMEM((BPW,), jnp.int32),
            pltpu.VMEM((BPW,), jnp.int32),
            pltpu.VMEM((BPW, DIM), jnp.float32),
            pltpu.VMEM((BPW, DIM), jnp.float32),
            pltpu.VMEM((BPW, DIM), jnp.float32),
            pltpu.VMEM((BPW,), jnp.float32),
            pltpu.SemaphoreType.DMA,
        ],
    )(_fmc_body)
    return run(prev, pos, neg, item_in_weight, item_out_weight)


def kernel(prev, pos, neg, item_in_weight, item_out_weight):
    return _fmc(prev.astype(jnp.int32), pos.astype(jnp.int32),
                neg.astype(jnp.int32), item_in_weight, item_out_weight)


# speedup vs baseline: 1.0640x; 1.0640x over previous
import functools

import jax
import jax.numpy as jnp
from jax import lax
from jax.experimental import pallas as pl
from jax.experimental.pallas import tpu as pltpu
from jax.experimental.pallas import tpu_sc as plsc

DIM = 64
PDIM = 128
BATCH = 16384
NUM_CORES = 2
NUM_SUBCORES = 16
LANES = 16
NW = NUM_CORES * NUM_SUBCORES
BPW = BATCH // NW
CHUNK = 128
NCHUNK = BPW // CHUNK
GPC = CHUNK // LANES


def _fmc_body(prev_hbm, pos_hbm, neg_hbm, win_hbm, wout_hbm, out_hbm,
              pidx, aidx, bidx, h0, h1, p0, p1, n0, n1, outv, sem0, sem1):
    wid = lax.axis_index("s") * NUM_CORES + lax.axis_index("c")
    base = wid * BPW

    pltpu.sync_copy(prev_hbm.at[pl.ds(base, BPW)], pidx)
    pltpu.sync_copy(pos_hbm.at[pl.ds(base, BPW)], aidx)
    pltpu.sync_copy(neg_hbm.at[pl.ds(base, BPW)], bidx)

    hbufs = (h0, h1)
    pbufs = (p0, p1)
    nbufs = (n0, n1)
    sems = (sem0, sem1)

    def fire(c):
        sl = pl.ds(c * CHUNK, CHUNK)
        sem = sems[c % 2]
        cps = (
            pltpu.make_async_copy(win_hbm.at[pidx.at[sl]], hbufs[c % 2], sem),
            pltpu.make_async_copy(wout_hbm.at[aidx.at[sl]], pbufs[c % 2], sem),
            pltpu.make_async_copy(wout_hbm.at[bidx.at[sl]], nbufs[c % 2], sem),
        )
        for cp in cps:
            cp.start()
        return cps

    iota = lax.iota(jnp.int32, LANES)
    inflight = fire(0)
    for c in range(NCHUNK):
        for cp in inflight:
            cp.wait()
        if c + 1 < NCHUNK:
            inflight = fire(c + 1)
        hb, pb, nb = hbufs[c % 2], pbufs[c % 2], nbufs[c % 2]

        def group(g, _):
            rows = g * LANES + iota
            acc = jnp.zeros((LANES,), jnp.float32)
            for d in range(DIM):
                col = jnp.full((LANES,), d, jnp.int32)
                hv = plsc.load_gather(hb, [rows, col])
                pv = plsc.load_gather(pb, [rows, col])
                nv = plsc.load_gather(nb, [rows, col])
                acc = acc + hv * (pv - nv)
            outv[pl.ds(c * CHUNK + g * LANES, LANES)] = acc
            return 0

        lax.fori_loop(0, GPC, group, 0)

    pltpu.sync_copy(outv, out_hbm.at[pl.ds(base, BPW)])


@jax.jit
def _fmc(prev, pos, neg, item_in_weight, item_out_weight):
    win = jnp.pad(item_in_weight, ((0, 0), (0, PDIM - DIM)))
    wout = jnp.pad(item_out_weight, ((0, 0), (0, PDIM - DIM)))
    mesh = plsc.VectorSubcoreMesh(core_axis_name="c", subcore_axis_name="s")
    run = functools.partial(
        pl.kernel,
        out_type=jax.ShapeDtypeStruct((BATCH,), jnp.float32),
        mesh=mesh,
        compiler_params=pltpu.CompilerParams(needs_layout_passes=False),
        scratch_types=[
            pltpu.VMEM((BPW,), jnp.int32),
            pltpu.VMEM((BPW,), jnp.int32),
            pltpu.VMEM((BPW,), jnp.int32),
            pltpu.VMEM((CHUNK, PDIM), jnp.float32),
            pltpu.VMEM((CHUNK, PDIM), jnp.float32),
            pltpu.VMEM((CHUNK, PDIM), jnp.float32),
            pltpu.VMEM((CHUNK, PDIM), jnp.float32),
            pltpu.VMEM((CHUNK, PDIM), jnp.float32),
            pltpu.VMEM((CHUNK, PDIM), jnp.float32),
            pltpu.VMEM((BPW,), jnp.float32),
            pltpu.SemaphoreType.DMA,
            pltpu.SemaphoreType.DMA,
        ],
    )(_fmc_body)
    return run(prev, pos, neg, win, wout)


def kernel(prev, pos, neg, item_in_weight, item_out_weight):
    return _fmc(prev.astype(jnp.int32), pos.astype(jnp.int32),
                neg.astype(jnp.int32), item_in_weight, item_out_weight)


# speedup vs baseline: 1.7932x; 1.6854x over previous
import functools

import jax
import jax.numpy as jnp
from jax import lax
from jax.experimental import pallas as pl
from jax.experimental.pallas import tpu as pltpu
from jax.experimental.pallas import tpu_sc as plsc

DIM = 64
PDIM = 128
BATCH = 16384
NUM_CORES = 2
NUM_SUBCORES = 16
LANES = 16
NW = NUM_CORES * NUM_SUBCORES
BPW = BATCH // NW
NITEMS = 1000001
BLKW = 128
BPWORKER = 245
RNG = BPWORKER * BLKW
NFULL = (NITEMS // BLKW)
STAG = 64
CHUNK = 64
NCHUNK = BPW // CHUNK
GPC = CHUNK // LANES


def _gather_pass(table, tail, scratch, vals, mlist, sortl, win0, win1, stag,
                 sidx, counts, offs, offs2, semw0, semw1, nstream, dump_base,
                 lo, hi, blk0, nblk, is_last):
    n = nstream * BATCH
    iota = lax.iota(jnp.int32, LANES)
    wins = (win0, win1)
    semws = (semw0, semw1)

    vals[pl.ds(n, LANES)] = jnp.full((LANES,), 2 * NITEMS, jnp.int32)

    def scan_step(i, cnt):
        v = vals[pl.ds(i * LANES, LANES)]
        posv = i * LANES + iota
        m = (v >= lo) & (v < hi)
        plsc.store_compressed(mlist.at[pl.ds(cnt, LANES)], posv, mask=m)
        return cnt + plsc.all_reduce_population_count(m)[0]

    cnt = lax.fori_loop(0, n // LANES, scan_step, jnp.int32(0))
    mlist[pl.ds(cnt, LANES)] = jnp.full((LANES,), n, jnp.int32)
    cnt16 = (cnt + LANES - 1) // LANES

    def zero_step(b, _):
        counts[b] = 0
        return 0

    lax.fori_loop(0, 256, zero_step, 0)

    def count_step(i, _):
        mvec = mlist[pl.ds(i * LANES, LANES)]
        vv = plsc.load_gather(vals, [mvec])
        bv = jnp.minimum((vv - lo) >> 7, 255)
        for j in range(LANES):
            counts[bv[j]] = counts[bv[j]] + 1
        return 0

    lax.fori_loop(0, cnt16, count_step, 0)

    offs[0] = 0
    offs2[0] = 0

    def prefix(b, _):
        t = offs[b] + counts[b]
        offs[b + 1] = t
        offs2[b + 1] = t
        return 0

    lax.fori_loop(0, 256, prefix, 0)

    def place_step(i, _):
        mvec = mlist[pl.ds(i * LANES, LANES)]
        vv = plsc.load_gather(vals, [mvec])
        bv = jnp.minimum((vv - lo) >> 7, 255)
        for j in range(LANES):
            b = bv[j]
            o = offs[b]
            plsc.store_scatter(sortl, [jnp.full((LANES,), o, jnp.int32)],
                               jnp.full((LANES,), mvec[j], jnp.int32),
                               mask=iota == 0)
            offs[b] = o + 1
        return 0

    lax.fori_loop(0, cnt16, place_step, 0)

    def fill_dumps():
        for q in range(STAG // LANES):
            sidx[pl.ds(q * LANES, LANES)] = dump_base + q * LANES + iota

    fill_dumps()

    def fire(b, par):
        bc = jnp.minimum(b, nblk - 1)
        cp = pltpu.make_async_copy(
            table.at[:, pl.ds((blk0 + bc) * BLKW, BLKW)], wins[par], semws[par])
        cp.start()

    def wait_win(par):
        pltpu.make_async_copy(
            table.at[:, pl.ds(0, BLKW)], wins[par], semws[par]).wait()

    def extract_entry(e, r, win):
        pe = sortl[pl.ds(e, LANES)][0]
        vv = vals[pl.ds(pe, LANES)][0]
        lane = jnp.full((LANES,), vv & (BLKW - 1), jnp.int32)
        for q in range(DIM // LANES):
            g = plsc.load_gather(win, [q * LANES + iota, lane])
            stag[r, pl.ds(q * LANES, LANES)] = g
        plsc.store_scatter(sidx, [jnp.full((LANES,), r, jnp.int32)],
                           jnp.full((LANES,), pe, jnp.int32),
                           mask=iota == 0)
        return r + 1

    def flush():
        pltpu.sync_copy(stag, scratch.at[sidx])
        fill_dumps()

    def proc_block(b, r, win):
        e0 = offs2[b]
        e1 = offs2[b + 1]

        def entry_step(e, rr):
            rr2 = extract_entry(e, rr, win)

            def do_flush(_):
                flush()
                return jnp.int32(0)

            return lax.cond(rr2 == STAG, do_flush, lambda _: rr2, 0)

        return lax.fori_loop(e0, e1, entry_step, r)

    fire(0, 0)

    def pair_step(i, r):
        b0 = 2 * i
        b1 = 2 * i + 1
        wait_win(0)
        fire(b0 + 1, 1)
        r = lax.cond(b0 < nblk,
                     lambda rr: proc_block(b0, rr, win0),
                     lambda rr: rr, r)
        wait_win(1)
        fire(b1 + 1, 0)
        r = lax.cond(b1 < nblk,
                     lambda rr: proc_block(b1, rr, win1),
                     lambda rr: rr, r)
        return r

    r = lax.fori_loop(0, (BPWORKER + 1) // 2, pair_step, jnp.int32(0))
    wait_win(0)

    @pl.when(is_last)
    def _tail():
        pltpu.sync_copy(tail, win0)
        e0 = offs2[nblk]
        e1 = offs2[nblk + 1]

        def entry_step(e, rr):
            rr2 = extract_entry(e, rr, win0)

            def do_flush(_):
                flush()
                return jnp.int32(0)

            return lax.cond(rr2 == STAG, do_flush, lambda _: rr2, 0)

        rt = lax.fori_loop(e0, e1, entry_step, r)

        @pl.when(rt > 0)
        def _():
            flush()

    @pl.when(jnp.logical_not(is_last) & (r > 0))
    def _final():
        flush()


def _scan_body(prev_hbm, pos_hbm, neg_hbm, wint_hbm, woutt_hbm,
               tin_hbm, tout_hbm, scrh_hbm, scrab_hbm,
               vals, mlist, sortl, win0, win1, stag, sidx,
               counts, offs, offs2, semw0, semw1):
    wid = lax.axis_index("s") * NUM_CORES + lax.axis_index("c")
    lo = wid * RNG
    hi = jnp.minimum(lo + RNG, NITEMS)
    blk0 = wid * BPWORKER
    nblk = jnp.minimum(BPWORKER, NFULL - blk0)
    is_last = wid == NW - 1

    pltpu.sync_copy(prev_hbm, vals.at[pl.ds(0, BATCH)])
    _gather_pass(wint_hbm, tin_hbm, scrh_hbm, vals, mlist, sortl, win0, win1,
                 stag, sidx, counts, offs, offs2, semw0, semw1, 1, BATCH,
                 lo, hi, blk0, nblk, is_last)

    pltpu.sync_copy(pos_hbm, vals.at[pl.ds(0, BATCH)])
    pltpu.sync_copy(neg_hbm, vals.at[pl.ds(BATCH, BATCH)])
    _gather_pass(woutt_hbm, tout_hbm, scrab_hbm, vals, mlist, sortl, win0,
                 win1, stag, sidx, counts, offs, offs2, semw0, semw1, 2,
                 2 * BATCH, lo, hi, blk0, nblk, is_last)


def _dot_body(scrh_hbm, scrab_hbm, out_hbm,
              h0, h1, p0, p1, n0, n1, outv, sem0, sem1):
    wid = lax.axis_index("s") * NUM_CORES + lax.axis_index("c")
    base = wid * BPW
    hbufs = (h0, h1)
    pbufs = (p0, p1)
    nbufs = (n0, n1)
    sems = (sem0, sem1)

    def fire(c):
        sl = pl.ds(base + c * CHUNK, CHUNK)
        sl2 = pl.ds(BATCH + base + c * CHUNK, CHUNK)
        sem = sems[c % 2]
        cps = (
            pltpu.make_async_copy(scrh_hbm.at[sl], hbufs[c % 2], sem),
            pltpu.make_async_copy(scrab_hbm.at[sl], pbufs[c % 2], sem),
            pltpu.make_async_copy(scrab_hbm.at[sl2], nbufs[c % 2], sem),
        )
        for cp in cps:
            cp.start()
        return cps

    iota = lax.iota(jnp.int32, LANES)
    inflight = fire(0)
    for c in range(NCHUNK):
        for cp in inflight:
            cp.wait()
        if c + 1 < NCHUNK:
            inflight = fire(c + 1)
        hb, pb, nb = hbufs[c % 2], pbufs[c % 2], nbufs[c % 2]

        def group(g, _):
            rows = g * LANES + iota
            acc = jnp.zeros((LANES,), jnp.float32)
            for d in range(DIM):
                col = jnp.full((LANES,), d, jnp.int32)
                hv = plsc.load_gather(hb, [rows, col])
                pv = plsc.load_gather(pb, [rows, col])
                nv = plsc.load_gather(nb, [rows, col])
                acc = acc + hv * (pv - nv)
            outv[pl.ds(c * CHUNK + g * LANES, LANES)] = acc
            return 0

        lax.fori_loop(0, GPC, group, 0)

    pltpu.sync_copy(outv, out_hbm.at[pl.ds(base, BPW)])


@jax.jit
def _fmc(prev, pos, neg, item_in_weight, item_out_weight):
    wint = item_in_weight.T
    woutt = item_out_weight.T
    pad = ((0, 0), (0, BLKW - (NITEMS - NFULL * BLKW)))
    tin = jnp.pad(item_in_weight[NFULL * BLKW:].T, pad)
    tout = jnp.pad(item_out_weight[NFULL * BLKW:].T, pad)
    mesh = plsc.VectorSubcoreMesh(core_axis_name="c", subcore_axis_name="s")

    scan = functools.partial(
        pl.kernel,
        out_type=(
            jax.ShapeDtypeStruct((BATCH + STAG, PDIM), jnp.float32),
            jax.ShapeDtypeStruct((2 * BATCH + STAG, PDIM), jnp.float32),
        ),
        mesh=mesh,
        compiler_params=pltpu.CompilerParams(needs_layout_passes=False),
        scratch_types=[
            pltpu.VMEM((2 * BATCH + LANES,), jnp.int32),
            pltpu.VMEM((2 * BATCH + 2 * LANES,), jnp.int32),
            pltpu.VMEM((2 * BATCH + 2 * LANES,), jnp.int32),
            pltpu.VMEM((DIM, BLKW), jnp.float32),
            pltpu.VMEM((DIM, BLKW), jnp.float32),
            pltpu.VMEM((STAG, PDIM), jnp.float32),
            pltpu.VMEM((STAG,), jnp.int32),
            pltpu.SMEM((258,), jnp.int32),
            pltpu.SMEM((258,), jnp.int32),
            pltpu.SMEM((258,), jnp.int32),
            pltpu.SemaphoreType.DMA,
            pltpu.SemaphoreType.DMA,
        ],
    )(_scan_body)
    scrh, scrab = scan(prev, pos, neg, wint, woutt, tin, tout)

    dot = functools.partial(
        pl.kernel,
        out_type=jax.ShapeDtypeStruct((BATCH,), jnp.float32),
        mesh=mesh,
        compiler_params=pltpu.CompilerParams(needs_layout_passes=False),
        scratch_types=[
            pltpu.VMEM((CHUNK, PDIM), jnp.float32),
            pltpu.VMEM((CHUNK, PDIM), jnp.float32),
            pltpu.VMEM((CHUNK, PDIM), jnp.float32),
            pltpu.VMEM((CHUNK, PDIM), jnp.float32),
            pltpu.VMEM((CHUNK, PDIM), jnp.float32),
            pltpu.VMEM((CHUNK, PDIM), jnp.float32),
            pltpu.VMEM((BPW,), jnp.float32),
            pltpu.SemaphoreType.DMA,
            pltpu.SemaphoreType.DMA,
        ],
    )(_dot_body)
    return dot(scrh, scrab)


def kernel(prev, pos, neg, item_in_weight, item_out_weight):
    return _fmc(prev.astype(jnp.int32), pos.astype(jnp.int32),
                neg.astype(jnp.int32), item_in_weight, item_out_weight)


# speedup vs baseline: 2.6750x; 1.4917x over previous
import functools

import jax
import jax.numpy as jnp
from jax import lax
from jax.experimental import pallas as pl
from jax.experimental.pallas import tpu as pltpu
from jax.experimental.pallas import tpu_sc as plsc

DIM = 64
PDIM = 128
BATCH = 16384
NUM_CORES = 2
NUM_SUBCORES = 16
LANES = 16
NW = NUM_CORES * NUM_SUBCORES
BPW = BATCH // NW
NITEMS = 1000001
BLKW = 128
BPWORKER = 245
RNG = BPWORKER * BLKW
NFULL = (NITEMS // BLKW)
STAG = 32
NBUF = 3
CHUNK = 64
NCHUNK = BPW // CHUNK
GPC = CHUNK // LANES


def _gather_pass(table, tail, scratch, vals, mlist, sortl, win0, win1, win2,
                 stag, sidx, counts, offs, offs2, semw0, semw1, semw2,
                 nstream, dump_base, lo, hi, blk0, nblk, is_last):
    n = nstream * BATCH
    iota = lax.iota(jnp.int32, LANES)
    wins = (win0, win1, win2)
    semws = (semw0, semw1, semw2)

    vals[pl.ds(n, LANES)] = jnp.full((LANES,), 2 * NITEMS, jnp.int32)

    def scan_step(i, cnt):
        v = vals[pl.ds(i * LANES, LANES)]
        posv = i * LANES + iota
        m = (v >= lo) & (v < hi)
        plsc.store_compressed(mlist.at[pl.ds(cnt, LANES)], posv, mask=m)
        return cnt + plsc.all_reduce_population_count(m)[0]

    cnt = lax.fori_loop(0, n // LANES, scan_step, jnp.int32(0))
    mlist[pl.ds(cnt, LANES)] = jnp.full((LANES,), n, jnp.int32)
    cnt16 = (cnt + LANES - 1) // LANES

    def zero_step(b, _):
        counts[b] = 0
        return 0

    lax.fori_loop(0, 256, zero_step, 0)

    def count_step(i, _):
        mvec = mlist[pl.ds(i * LANES, LANES)]
        vv = plsc.load_gather(vals, [mvec])
        bv = jnp.minimum((vv - lo) >> 7, 255)
        for j in range(LANES):
            counts[bv[j]] = counts[bv[j]] + 1
        return 0

    lax.fori_loop(0, cnt16, count_step, 0)

    offs[0] = 0
    offs2[0] = 0

    def prefix(b, _):
        t = offs[b] + counts[b]
        offs[b + 1] = t
        offs2[b + 1] = t
        return 0

    lax.fori_loop(0, 256, prefix, 0)

    def place_step(i, _):
        mvec = mlist[pl.ds(i * LANES, LANES)]
        vv = plsc.load_gather(vals, [mvec])
        bv = jnp.minimum((vv - lo) >> 7, 255)
        for j in range(LANES):
            b = bv[j]
            o = offs[b]
            plsc.store_scatter(sortl, [jnp.full((LANES,), o, jnp.int32)],
                               jnp.full((LANES,), mvec[j], jnp.int32),
                               mask=iota == 0)
            offs[b] = o + 1
        return 0

    lax.fori_loop(0, cnt16, place_step, 0)

    def fill_dumps():
        for q in range(STAG // LANES):
            sidx[pl.ds(q * LANES, LANES)] = dump_base + q * LANES + iota

    fill_dumps()

    def fire(b, par):
        bc = jnp.minimum(b, nblk - 1)
        cp = pltpu.make_async_copy(
            table.at[:, pl.ds((blk0 + bc) * BLKW, BLKW)], wins[par], semws[par])
        cp.start()

    def wait_win(par):
        pltpu.make_async_copy(
            table.at[:, pl.ds(0, BLKW)], wins[par], semws[par]).wait()

    def extract_entry(e, r, win):
        pe = sortl[pl.ds(e, LANES)][0]
        vv = vals[pl.ds(pe, LANES)][0]
        lane = jnp.full((LANES,), vv & (BLKW - 1), jnp.int32)
        for q in range(DIM // LANES):
            g = plsc.load_gather(win, [q * LANES + iota, lane])
            stag[r, pl.ds(q * LANES, LANES)] = g
        plsc.store_scatter(sidx, [jnp.full((LANES,), r, jnp.int32)],
                           jnp.full((LANES,), pe, jnp.int32),
                           mask=iota == 0)
        return r + 1

    def flush():
        pltpu.sync_copy(stag, scratch.at[sidx])
        fill_dumps()

    def proc_block(b, r, win):
        e0 = offs2[b]
        e1 = offs2[b + 1]

        def entry_step(e, rr):
            rr2 = extract_entry(e, rr, win)

            def do_flush(_):
                flush()
                return jnp.int32(0)

            return lax.cond(rr2 == STAG, do_flush, lambda _: rr2, 0)

        return lax.fori_loop(e0, e1, entry_step, r)

    fire(0, 0)
    fire(1, 1)

    def tri_step(i, r):
        for k in range(NBUF):
            b = NBUF * i + k
            wait_win(k)
            fire(b + 2, (k + 2) % NBUF)
            r = lax.cond(b < nblk,
                         functools.partial(proc_block, b, win=wins[k]),
                         lambda rr: rr, r)
        return r

    ntri = (BPWORKER + NBUF) // NBUF
    r = lax.fori_loop(0, ntri, tri_step, jnp.int32(0))
    wait_win((NBUF * ntri) % NBUF)
    wait_win((NBUF * ntri + 1) % NBUF)

    @pl.when(is_last)
    def _tail():
        pltpu.sync_copy(tail, win0)
        e0 = offs2[nblk]
        e1 = offs2[nblk + 1]

        def entry_step(e, rr):
            rr2 = extract_entry(e, rr, win0)

            def do_flush(_):
                flush()
                return jnp.int32(0)

            return lax.cond(rr2 == STAG, do_flush, lambda _: rr2, 0)

        rt = lax.fori_loop(e0, e1, entry_step, r)

        @pl.when(rt > 0)
        def _():
            flush()

    @pl.when(jnp.logical_not(is_last) & (r > 0))
    def _final():
        flush()


def _scan_body(prev_hbm, pos_hbm, neg_hbm, wint_hbm, woutt_hbm,
               tin_hbm, tout_hbm, scrh_hbm, scrab_hbm,
               vals, mlist, sortl, win0, win1, win2, stag, sidx,
               counts, offs, offs2, semw0, semw1, semw2):
    wid = lax.axis_index("s") * NUM_CORES + lax.axis_index("c")
    lo = wid * RNG
    hi = jnp.minimum(lo + RNG, NITEMS)
    blk0 = wid * BPWORKER
    nblk = jnp.minimum(BPWORKER, NFULL - blk0)
    is_last = wid == NW - 1

    pltpu.sync_copy(prev_hbm, vals.at[pl.ds(0, BATCH)])
    _gather_pass(wint_hbm, tin_hbm, scrh_hbm, vals, mlist, sortl, win0, win1,
                 win2, stag, sidx, counts, offs, offs2, semw0, semw1, semw2,
                 1, BATCH, lo, hi, blk0, nblk, is_last)

    pltpu.sync_copy(pos_hbm, vals.at[pl.ds(0, BATCH)])
    pltpu.sync_copy(neg_hbm, vals.at[pl.ds(BATCH, BATCH)])
    _gather_pass(woutt_hbm, tout_hbm, scrab_hbm, vals, mlist, sortl, win0,
                 win1, win2, stag, sidx, counts, offs, offs2, semw0, semw1,
                 semw2, 2, 2 * BATCH, lo, hi, blk0, nblk, is_last)


def _dot_body(scrh_hbm, scrab_hbm, out_hbm,
              h0, h1, p0, p1, n0, n1, outv, sem0, sem1):
    wid = lax.axis_index("s") * NUM_CORES + lax.axis_index("c")
    base = wid * BPW
    hbufs = (h0, h1)
    pbufs = (p0, p1)
    nbufs = (n0, n1)
    sems = (sem0, sem1)

    def fire(c):
        sl = pl.ds(base + c * CHUNK, CHUNK)
        sl2 = pl.ds(BATCH + base + c * CHUNK, CHUNK)
        sem = sems[c % 2]
        cps = (
            pltpu.make_async_copy(scrh_hbm.at[sl], hbufs[c % 2], sem),
            pltpu.make_async_copy(scrab_hbm.at[sl], pbufs[c % 2], sem),
            pltpu.make_async_copy(scrab_hbm.at[sl2], nbufs[c % 2], sem),
        )
        for cp in cps:
            cp.start()
        return cps

    iota = lax.iota(jnp.int32, LANES)
    inflight = fire(0)
    for c in range(NCHUNK):
        for cp in inflight:
            cp.wait()
        if c + 1 < NCHUNK:
            inflight = fire(c + 1)
        hb, pb, nb = hbufs[c % 2], pbufs[c % 2], nbufs[c % 2]

        def group(g, _):
            rows = g * LANES + iota
            acc = jnp.zeros((LANES,), jnp.float32)
            for d in range(DIM):
                col = jnp.full((LANES,), d, jnp.int32)
                hv = plsc.load_gather(hb, [rows, col])
                pv = plsc.load_gather(pb, [rows, col])
                nv = plsc.load_gather(nb, [rows, col])
                acc = acc + hv * (pv - nv)
            outv[pl.ds(c * CHUNK + g * LANES, LANES)] = acc
            return 0

        lax.fori_loop(0, GPC, group, 0)

    pltpu.sync_copy(outv, out_hbm.at[pl.ds(base, BPW)])


@jax.jit
def _fmc(prev, pos, neg, item_in_weight, item_out_weight):
    wint = item_in_weight.T
    woutt = item_out_weight.T
    pad = ((0, 0), (0, BLKW - (NITEMS - NFULL * BLKW)))
    tin = jnp.pad(item_in_weight[NFULL * BLKW:].T, pad)
    tout = jnp.pad(item_out_weight[NFULL * BLKW:].T, pad)
    mesh = plsc.VectorSubcoreMesh(core_axis_name="c", subcore_axis_name="s")

    scan = functools.partial(
        pl.kernel,
        out_type=(
            jax.ShapeDtypeStruct((BATCH + STAG, PDIM), jnp.float32),
            jax.ShapeDtypeStruct((2 * BATCH + STAG, PDIM), jnp.float32),
        ),
        mesh=mesh,
        compiler_params=pltpu.CompilerParams(needs_layout_passes=False),
        scratch_types=[
            pltpu.VMEM((2 * BATCH + LANES,), jnp.int32),
            pltpu.VMEM((2 * BATCH + 2 * LANES,), jnp.int32),
            pltpu.VMEM((2 * BATCH + 2 * LANES,), jnp.int32),
            pltpu.VMEM((DIM, BLKW), jnp.float32),
            pltpu.VMEM((DIM, BLKW), jnp.float32),
            pltpu.VMEM((DIM, BLKW), jnp.float32),
            pltpu.VMEM((STAG, PDIM), jnp.float32),
            pltpu.VMEM((STAG,), jnp.int32),
            pltpu.SMEM((258,), jnp.int32),
            pltpu.SMEM((258,), jnp.int32),
            pltpu.SMEM((258,), jnp.int32),
            pltpu.SemaphoreType.DMA,
            pltpu.SemaphoreType.DMA,
            pltpu.SemaphoreType.DMA,
        ],
    )(_scan_body)
    scrh, scrab = scan(prev, pos, neg, wint, woutt, tin, tout)

    dot = functools.partial(
        pl.kernel,
        out_type=jax.ShapeDtypeStruct((BATCH,), jnp.float32),
        mesh=mesh,
        compiler_params=pltpu.CompilerParams(needs_layout_passes=False),
        scratch_types=[
            pltpu.VMEM((CHUNK, PDIM), jnp.float32),
            pltpu.VMEM((CHUNK, PDIM), jnp.float32),
            pltpu.VMEM((CHUNK, PDIM), jnp.float32),
            pltpu.VMEM((CHUNK, PDIM), jnp.float32),
            pltpu.VMEM((CHUNK, PDIM), jnp.float32),
            pltpu.VMEM((CHUNK, PDIM), jnp.float32),
            pltpu.VMEM((BPW,), jnp.float32),
            pltpu.SemaphoreType.DMA,
            pltpu.SemaphoreType.DMA,
        ],
    )(_dot_body)
    return dot(scrh, scrab)


def kernel(prev, pos, neg, item_in_weight, item_out_weight):
    return _fmc(prev.astype(jnp.int32), pos.astype(jnp.int32),
                neg.astype(jnp.int32), item_in_weight, item_out_weight)


# speedup vs baseline: 2.6965x; 1.0080x over previous
import functools

import jax
import jax.numpy as jnp
from jax import lax
from jax.experimental import pallas as pl
from jax.experimental.pallas import tpu as pltpu
from jax.experimental.pallas import tpu_sc as plsc

DIM = 64
PDIM = 128
BATCH = 16384
NUM_CORES = 2
NUM_SUBCORES = 16
LANES = 16
NW = NUM_CORES * NUM_SUBCORES
BPW = BATCH // NW
NITEMS = 1000001
BLKW = 128
BPWORKER = 245
RNG = BPWORKER * BLKW
NFULL = (NITEMS // BLKW)
STAG = 32
NBUF = 3
CHUNK = 128
NCHUNK = BPW // CHUNK
GPC = CHUNK // LANES


def _gather_pass(table, tail, scratch, vals, mlist, sortl, win0, win1, win2,
                 stag, sidx, counts, offs, offs2, semw0, semw1, semw2,
                 nstream, dump_base, lo, hi, blk0, nblk, is_last):
    n = nstream * BATCH
    iota = lax.iota(jnp.int32, LANES)
    wins = (win0, win1, win2)
    semws = (semw0, semw1, semw2)

    vals[pl.ds(n, LANES)] = jnp.full((LANES,), 2 * NITEMS, jnp.int32)

    def scan_step(i, cnt):
        v = vals[pl.ds(i * LANES, LANES)]
        posv = i * LANES + iota
        m = (v >= lo) & (v < hi)
        plsc.store_compressed(mlist.at[pl.ds(cnt, LANES)], posv, mask=m)
        return cnt + plsc.all_reduce_population_count(m)[0]

    cnt = lax.fori_loop(0, n // LANES, scan_step, jnp.int32(0))
    mlist[pl.ds(cnt, LANES)] = jnp.full((LANES,), n, jnp.int32)
    cnt16 = (cnt + LANES - 1) // LANES

    def zero_step(b, _):
        counts[b] = 0
        return 0

    lax.fori_loop(0, 256, zero_step, 0)

    def count_step(i, _):
        mvec = mlist[pl.ds(i * LANES, LANES)]
        vv = plsc.load_gather(vals, [mvec])
        bv = jnp.minimum((vv - lo) >> 7, 255)
        for j in range(LANES):
            counts[bv[j]] = counts[bv[j]] + 1
        return 0

    lax.fori_loop(0, cnt16, count_step, 0)

    offs[0] = 0
    offs2[0] = 0

    def prefix(b, _):
        t = offs[b] + counts[b]
        offs[b + 1] = t
        offs2[b + 1] = t
        return 0

    lax.fori_loop(0, 256, prefix, 0)

    def place_step(i, _):
        mvec = mlist[pl.ds(i * LANES, LANES)]
        vv = plsc.load_gather(vals, [mvec])
        bv = jnp.minimum((vv - lo) >> 7, 255)
        for j in range(LANES):
            b = bv[j]
            o = offs[b]
            plsc.store_scatter(sortl, [jnp.full((LANES,), o, jnp.int32)],
                               jnp.full((LANES,), mvec[j], jnp.int32),
                               mask=iota == 0)
            offs[b] = o + 1
        return 0

    lax.fori_loop(0, cnt16, place_step, 0)

    def fill_dumps():
        for q in range(STAG // LANES):
            sidx[pl.ds(q * LANES, LANES)] = dump_base + q * LANES + iota

    fill_dumps()

    def fire(b, par):
        bc = jnp.minimum(b, nblk - 1)
        cp = pltpu.make_async_copy(
            table.at[:, pl.ds((blk0 + bc) * BLKW, BLKW)], wins[par], semws[par])
        cp.start()

    def wait_win(par):
        pltpu.make_async_copy(
            table.at[:, pl.ds(0, BLKW)], wins[par], semws[par]).wait()

    def extract_entry(e, r, win):
        pe = sortl[pl.ds(e, LANES)][0]
        vv = vals[pl.ds(pe, LANES)][0]
        lane = jnp.full((LANES,), vv & (BLKW - 1), jnp.int32)
        for q in range(DIM // LANES):
            g = plsc.load_gather(win, [q * LANES + iota, lane])
            stag[r, pl.ds(q * LANES, LANES)] = g
        plsc.store_scatter(sidx, [jnp.full((LANES,), r, jnp.int32)],
                           jnp.full((LANES,), pe, jnp.int32),
                           mask=iota == 0)
        return r + 1

    def flush():
        pltpu.sync_copy(stag, scratch.at[sidx])
        fill_dumps()

    def proc_block(b, r, win):
        e0 = offs2[b]
        e1 = offs2[b + 1]

        def entry_step(e, rr):
            rr2 = extract_entry(e, rr, win)

            def do_flush(_):
                flush()
                return jnp.int32(0)

            return lax.cond(rr2 == STAG, do_flush, lambda _: rr2, 0)

        return lax.fori_loop(e0, e1, entry_step, r)

    fire(0, 0)
    fire(1, 1)

    def tri_step(i, r):
        for k in range(NBUF):
            b = NBUF * i + k
            wait_win(k)
            fire(b + 2, (k + 2) % NBUF)
            r = lax.cond(b < nblk,
                         functools.partial(proc_block, b, win=wins[k]),
                         lambda rr: rr, r)
        return r

    ntri = (BPWORKER + NBUF) // NBUF
    r = lax.fori_loop(0, ntri, tri_step, jnp.int32(0))
    wait_win((NBUF * ntri) % NBUF)
    wait_win((NBUF * ntri + 1) % NBUF)

    @pl.when(is_last)
    def _tail():
        pltpu.sync_copy(tail, win0)
        e0 = offs2[nblk]
        e1 = offs2[nblk + 1]

        def entry_step(e, rr):
            rr2 = extract_entry(e, rr, win0)

            def do_flush(_):
                flush()
                return jnp.int32(0)

            return lax.cond(rr2 == STAG, do_flush, lambda _: rr2, 0)

        rt = lax.fori_loop(e0, e1, entry_step, r)

        @pl.when(rt > 0)
        def _():
            flush()

    @pl.when(jnp.logical_not(is_last) & (r > 0))
    def _final():
        flush()


def _scan_body(prev_hbm, pos_hbm, neg_hbm, wint_hbm, woutt_hbm,
               tin_hbm, tout_hbm, scrh_hbm, scrab_hbm,
               vals, mlist, sortl, win0, win1, win2, stag, sidx,
               counts, offs, offs2, semw0, semw1, semw2):
    wid = lax.axis_index("s") * NUM_CORES + lax.axis_index("c")
    lo = wid * RNG
    hi = jnp.minimum(lo + RNG, NITEMS)
    blk0 = wid * BPWORKER
    nblk = jnp.minimum(BPWORKER, NFULL - blk0)
    is_last = wid == NW - 1

    pltpu.sync_copy(prev_hbm, vals.at[pl.ds(0, BATCH)])
    _gather_pass(wint_hbm, tin_hbm, scrh_hbm, vals, mlist, sortl, win0, win1,
                 win2, stag, sidx, counts, offs, offs2, semw0, semw1, semw2,
                 1, BATCH, lo, hi, blk0, nblk, is_last)

    pltpu.sync_copy(pos_hbm, vals.at[pl.ds(0, BATCH)])
    pltpu.sync_copy(neg_hbm, vals.at[pl.ds(BATCH, BATCH)])
    _gather_pass(woutt_hbm, tout_hbm, scrab_hbm, vals, mlist, sortl, win0,
                 win1, win2, stag, sidx, counts, offs, offs2, semw0, semw1,
                 semw2, 2, 2 * BATCH, lo, hi, blk0, nblk, is_last)


def _dot_body(scrh_hbm, scrab_hbm, out_hbm,
              h0, h1, p0, p1, n0, n1, outv, sem0, sem1):
    wid = lax.axis_index("s") * NUM_CORES + lax.axis_index("c")
    base = wid * BPW
    hbufs = (h0, h1)
    pbufs = (p0, p1)
    nbufs = (n0, n1)
    sems = (sem0, sem1)

    def fire(c):
        sl = pl.ds(base + c * CHUNK, CHUNK)
        sl2 = pl.ds(BATCH + base + c * CHUNK, CHUNK)
        sem = sems[c % 2]
        cps = (
            pltpu.make_async_copy(scrh_hbm.at[sl], hbufs[c % 2], sem),
            pltpu.make_async_copy(scrab_hbm.at[sl], pbufs[c % 2], sem),
            pltpu.make_async_copy(scrab_hbm.at[sl2], nbufs[c % 2], sem),
        )
        for cp in cps:
            cp.start()
        return cps

    iota = lax.iota(jnp.int32, LANES)
    inflight = fire(0)
    for c in range(NCHUNK):
        for cp in inflight:
            cp.wait()
        if c + 1 < NCHUNK:
            inflight = fire(c + 1)
        hb, pb, nb = hbufs[c % 2], pbufs[c % 2], nbufs[c % 2]

        def group(g, _):
            rows = g * LANES + iota
            acc = jnp.zeros((LANES,), jnp.float32)
            for d in range(DIM):
                col = jnp.full((LANES,), d, jnp.int32)
                hv = plsc.load_gather(hb, [rows, col])
                pv = plsc.load_gather(pb, [rows, col])
                nv = plsc.load_gather(nb, [rows, col])
                acc = acc + hv * (pv - nv)
            outv[pl.ds(c * CHUNK + g * LANES, LANES)] = acc
            return 0

        lax.fori_loop(0, GPC, group, 0)

    pltpu.sync_copy(outv, out_hbm.at[pl.ds(base, BPW)])


@jax.jit
def _fmc(prev, pos, neg, item_in_weight, item_out_weight):
    wint = item_in_weight.T
    woutt = item_out_weight.T
    pad = ((0, 0), (0, BLKW - (NITEMS - NFULL * BLKW)))
    tin = jnp.pad(item_in_weight[NFULL * BLKW:].T, pad)
    tout = jnp.pad(item_out_weight[NFULL * BLKW:].T, pad)
    mesh = plsc.VectorSubcoreMesh(core_axis_name="c", subcore_axis_name="s")

    scan = functools.partial(
        pl.kernel,
        out_type=(
            jax.ShapeDtypeStruct((BATCH + STAG, PDIM), jnp.float32),
            jax.ShapeDtypeStruct((2 * BATCH + STAG, PDIM), jnp.float32),
        ),
        mesh=mesh,
        compiler_params=pltpu.CompilerParams(needs_layout_passes=False),
        scratch_types=[
            pltpu.VMEM((2 * BATCH + LANES,), jnp.int32),
            pltpu.VMEM((2 * BATCH + 2 * LANES,), jnp.int32),
            pltpu.VMEM((2 * BATCH + 2 * LANES,), jnp.int32),
            pltpu.VMEM((DIM, BLKW), jnp.float32),
            pltpu.VMEM((DIM, BLKW), jnp.float32),
            pltpu.VMEM((DIM, BLKW), jnp.float32),
            pltpu.VMEM((STAG, PDIM), jnp.float32),
            pltpu.VMEM((STAG,), jnp.int32),
            pltpu.SMEM((258,), jnp.int32),
            pltpu.SMEM((258,), jnp.int32),
            pltpu.SMEM((258,), jnp.int32),
            pltpu.SemaphoreType.DMA,
            pltpu.SemaphoreType.DMA,
            pltpu.SemaphoreType.DMA,
        ],
    )(_scan_body)
    scrh, scrab = scan(prev, pos, neg, wint, woutt, tin, tout)

    dot = functools.partial(
        pl.kernel,
        out_type=jax.ShapeDtypeStruct((BATCH,), jnp.float32),
        mesh=mesh,
        compiler_params=pltpu.CompilerParams(needs_layout_passes=False),
        scratch_types=[
            pltpu.VMEM((CHUNK, PDIM), jnp.float32),
            pltpu.VMEM((CHUNK, PDIM), jnp.float32),
            pltpu.VMEM((CHUNK, PDIM), jnp.float32),
            pltpu.VMEM((CHUNK, PDIM), jnp.float32),
            pltpu.VMEM((CHUNK, PDIM), jnp.float32),
            pltpu.VMEM((CHUNK, PDIM), jnp.float32),
            pltpu.VMEM((BPW,), jnp.float32),
            pltpu.SemaphoreType.DMA,
            pltpu.SemaphoreType.DMA,
        ],
    )(_dot_body)
    return dot(scrh, scrab)


def kernel(prev, pos, neg, item_in_weight, item_out_weight):
    return _fmc(prev.astype(jnp.int32), pos.astype(jnp.int32),
                neg.astype(jnp.int32), item_in_weight, item_out_weight)


# speedup vs baseline: 2.7103x; 1.0051x over previous
import functools

import jax
import jax.numpy as jnp
from jax import lax
from jax.experimental import pallas as pl
from jax.experimental.pallas import tpu as pltpu
from jax.experimental.pallas import tpu_sc as plsc

DIM = 64
PDIM = 128
BATCH = 16384
NUM_CORES = 2
NUM_SUBCORES = 16
LANES = 16
NW = NUM_CORES * NUM_SUBCORES
BPW = BATCH // NW
NITEMS = 1000001
BLKW = 128
BPWORKER = 245
RNG = BPWORKER * BLKW
NFULL = (NITEMS // BLKW)
STAG = 32
NBUF = 3
CHUNK = 128
NCHUNK = BPW // CHUNK
GPC = CHUNK // LANES


def _gather_pass(table, tail, scratch, vals, mlist, sortl, win0, win1, win2,
                 stag, sidx, counts, offs, offs2, semw0, semw1, semw2,
                 nstream, dump_base, lo, hi, blk0, nblk, is_last):
    n = nstream * BATCH
    iota = lax.iota(jnp.int32, LANES)
    wins = (win0, win1, win2)
    semws = (semw0, semw1, semw2)

    def fire(b, par):
        bc = jnp.minimum(b, nblk - 1)
        cp = pltpu.make_async_copy(
            table.at[:, pl.ds((blk0 + bc) * BLKW, BLKW)], wins[par], semws[par])
        cp.start()

    fire(0, 0)
    fire(1, 1)

    vals[pl.ds(n, LANES)] = jnp.full((LANES,), 2 * NITEMS, jnp.int32)

    def scan_step(i, cnt):
        v = vals[pl.ds(i * LANES, LANES)]
        posv = i * LANES + iota
        m = (v >= lo) & (v < hi)
        plsc.store_compressed(mlist.at[pl.ds(cnt, LANES)], posv, mask=m)
        return cnt + plsc.all_reduce_population_count(m)[0]

    cnt = lax.fori_loop(0, n // LANES, scan_step, jnp.int32(0))
    mlist[pl.ds(cnt, LANES)] = jnp.full((LANES,), n, jnp.int32)
    cnt16 = (cnt + LANES - 1) // LANES

    def zero_step(b, _):
        counts[b] = 0
        return 0

    lax.fori_loop(0, 256, zero_step, 0)

    def count_step(i, _):
        mvec = mlist[pl.ds(i * LANES, LANES)]
        vv = plsc.load_gather(vals, [mvec])
        bv = jnp.minimum((vv - lo) >> 7, 255)
        for j in range(LANES):
            counts[bv[j]] = counts[bv[j]] + 1
        return 0

    lax.fori_loop(0, cnt16, count_step, 0)

    offs[0] = 0
    offs2[0] = 0

    def prefix(b, _):
        t = offs[b] + counts[b]
        offs[b + 1] = t
        offs2[b + 1] = t
        return 0

    lax.fori_loop(0, 256, prefix, 0)

    def place_step(i, _):
        mvec = mlist[pl.ds(i * LANES, LANES)]
        vv = plsc.load_gather(vals, [mvec])
        bv = jnp.minimum((vv - lo) >> 7, 255)
        for j in range(LANES):
            b = bv[j]
            o = offs[b]
            plsc.store_scatter(sortl, [jnp.full((LANES,), o, jnp.int32)],
                               jnp.full((LANES,), mvec[j], jnp.int32),
                               mask=iota == 0)
            offs[b] = o + 1
        return 0

    lax.fori_loop(0, cnt16, place_step, 0)

    def fill_dumps():
        for q in range(STAG // LANES):
            sidx[pl.ds(q * LANES, LANES)] = dump_base + q * LANES + iota

    fill_dumps()

    def wait_win(par):
        pltpu.make_async_copy(
            table.at[:, pl.ds(0, BLKW)], wins[par], semws[par]).wait()

    def extract_entry(e, r, win):
        pe = sortl[pl.ds(e, LANES)][0]
        vv = vals[pl.ds(pe, LANES)][0]
        lane = jnp.full((LANES,), vv & (BLKW - 1), jnp.int32)
        for q in range(DIM // LANES):
            g = plsc.load_gather(win, [q * LANES + iota, lane])
            stag[r, pl.ds(q * LANES, LANES)] = g
        plsc.store_scatter(sidx, [jnp.full((LANES,), r, jnp.int32)],
                           jnp.full((LANES,), pe, jnp.int32),
                           mask=iota == 0)
        return r + 1

    def flush():
        pltpu.sync_copy(stag, scratch.at[sidx])
        fill_dumps()

    def proc_block(b, r, win):
        e0 = offs2[b]
        e1 = offs2[b + 1]

        def entry_step(e, rr):
            rr2 = extract_entry(e, rr, win)

            def do_flush(_):
                flush()
                return jnp.int32(0)

            return lax.cond(rr2 == STAG, do_flush, lambda _: rr2, 0)

        return lax.fori_loop(e0, e1, entry_step, r)

    def tri_step(i, r):
        for k in range(NBUF):
            b = NBUF * i + k
            wait_win(k)
            fire(b + 2, (k + 2) % NBUF)
            r = lax.cond(b < nblk,
                         functools.partial(proc_block, b, win=wins[k]),
                         lambda rr: rr, r)
        return r

    ntri = (BPWORKER + NBUF) // NBUF
    r = lax.fori_loop(0, ntri, tri_step, jnp.int32(0))
    wait_win((NBUF * ntri) % NBUF)
    wait_win((NBUF * ntri + 1) % NBUF)

    @pl.when(is_last)
    def _tail():
        pltpu.sync_copy(tail, win0)
        e0 = offs2[nblk]
        e1 = offs2[nblk + 1]

        def entry_step(e, rr):
            rr2 = extract_entry(e, rr, win0)

            def do_flush(_):
                flush()
                return jnp.int32(0)

            return lax.cond(rr2 == STAG, do_flush, lambda _: rr2, 0)

        rt = lax.fori_loop(e0, e1, entry_step, r)

        @pl.when(rt > 0)
        def _():
            flush()

    @pl.when(jnp.logical_not(is_last) & (r > 0))
    def _final():
        flush()


def _scan_body(prev_hbm, pos_hbm, neg_hbm, wint_hbm, woutt_hbm,
               tin_hbm, tout_hbm, scrh_hbm, scrab_hbm,
               vals, mlist, sortl, win0, win1, win2, stag, sidx,
               counts, offs, offs2, semw0, semw1, semw2):
    wid = lax.axis_index("s") * NUM_CORES + lax.axis_index("c")
    lo = wid * RNG
    hi = jnp.minimum(lo + RNG, NITEMS)
    blk0 = wid * BPWORKER
    nblk = jnp.minimum(BPWORKER, NFULL - blk0)
    is_last = wid == NW - 1

    pltpu.sync_copy(prev_hbm, vals.at[pl.ds(0, BATCH)])
    _gather_pass(wint_hbm, tin_hbm, scrh_hbm, vals, mlist, sortl, win0, win1,
                 win2, stag, sidx, counts, offs, offs2, semw0, semw1, semw2,
                 1, BATCH, lo, hi, blk0, nblk, is_last)

    pltpu.sync_copy(pos_hbm, vals.at[pl.ds(0, BATCH)])
    pltpu.sync_copy(neg_hbm, vals.at[pl.ds(BATCH, BATCH)])
    _gather_pass(woutt_hbm, tout_hbm, scrab_hbm, vals, mlist, sortl, win0,
                 win1, win2, stag, sidx, counts, offs, offs2, semw0, semw1,
                 semw2, 2, 2 * BATCH, lo, hi, blk0, nblk, is_last)


def _dot_body(scrh_hbm, scrab_hbm, out_hbm,
              h0, h1, p0, p1, n0, n1, outv, sem0, sem1):
    wid = lax.axis_index("s") * NUM_CORES + lax.axis_index("c")
    base = wid * BPW
    hbufs = (h0, h1)
    pbufs = (p0, p1)
    nbufs = (n0, n1)
    sems = (sem0, sem1)

    def fire(c):
        sl = pl.ds(base + c * CHUNK, CHUNK)
        sl2 = pl.ds(BATCH + base + c * CHUNK, CHUNK)
        sem = sems[c % 2]
        cps = (
            pltpu.make_async_copy(scrh_hbm.at[sl], hbufs[c % 2], sem),
            pltpu.make_async_copy(scrab_hbm.at[sl], pbufs[c % 2], sem),
            pltpu.make_async_copy(scrab_hbm.at[sl2], nbufs[c % 2], sem),
        )
        for cp in cps:
            cp.start()
        return cps

    iota = lax.iota(jnp.int32, LANES)
    inflight = fire(0)
    for c in range(NCHUNK):
        for cp in inflight:
            cp.wait()
        if c + 1 < NCHUNK:
            inflight = fire(c + 1)
        hb, pb, nb = hbufs[c % 2], pbufs[c % 2], nbufs[c % 2]

        def group(g, _):
            rows = g * LANES + iota
            acc = jnp.zeros((LANES,), jnp.float32)
            for d in range(DIM):
                col = jnp.full((LANES,), d, jnp.int32)
                hv = plsc.load_gather(hb, [rows, col])
                pv = plsc.load_gather(pb, [rows, col])
                nv = plsc.load_gather(nb, [rows, col])
                acc = acc + hv * (pv - nv)
            outv[pl.ds(c * CHUNK + g * LANES, LANES)] = acc
            return 0

        lax.fori_loop(0, GPC, group, 0)

    pltpu.sync_copy(outv, out_hbm.at[pl.ds(base, BPW)])


@jax.jit
def _fmc(prev, pos, neg, item_in_weight, item_out_weight):
    wint = item_in_weight.T
    woutt = item_out_weight.T
    pad = ((0, 0), (0, BLKW - (NITEMS - NFULL * BLKW)))
    tin = jnp.pad(item_in_weight[NFULL * BLKW:].T, pad)
    tout = jnp.pad(item_out_weight[NFULL * BLKW:].T, pad)
    mesh = plsc.VectorSubcoreMesh(core_axis_name="c", subcore_axis_name="s")

    scan = functools.partial(
        pl.kernel,
        out_type=(
            jax.ShapeDtypeStruct((BATCH + STAG, PDIM), jnp.float32),
            jax.ShapeDtypeStruct((2 * BATCH + STAG, PDIM), jnp.float32),
        ),
        mesh=mesh,
        compiler_params=pltpu.CompilerParams(needs_layout_passes=False),
        scratch_types=[
            pltpu.VMEM((2 * BATCH + LANES,), jnp.int32),
            pltpu.VMEM((2 * BATCH + 2 * LANES,), jnp.int32),
            pltpu.VMEM((2 * BATCH + 2 * LANES,), jnp.int32),
            pltpu.VMEM((DIM, BLKW), jnp.float32),
            pltpu.VMEM((DIM, BLKW), jnp.float32),
            pltpu.VMEM((DIM, BLKW), jnp.float32),
            pltpu.VMEM((STAG, PDIM), jnp.float32),
            pltpu.VMEM((STAG,), jnp.int32),
            pltpu.SMEM((258,), jnp.int32),
            pltpu.SMEM((258,), jnp.int32),
            pltpu.SMEM((258,), jnp.int32),
            pltpu.SemaphoreType.DMA,
            pltpu.SemaphoreType.DMA,
            pltpu.SemaphoreType.DMA,
        ],
    )(_scan_body)
    scrh, scrab = scan(prev, pos, neg, wint, woutt, tin, tout)

    dot = functools.partial(
        pl.kernel,
        out_type=jax.ShapeDtypeStruct((BATCH,), jnp.float32),
        mesh=mesh,
        compiler_params=pltpu.CompilerParams(needs_layout_passes=False),
        scratch_types=[
            pltpu.VMEM((CHUNK, PDIM), jnp.float32),
            pltpu.VMEM((CHUNK, PDIM), jnp.float32),
            pltpu.VMEM((CHUNK, PDIM), jnp.float32),
            pltpu.VMEM((CHUNK, PDIM), jnp.float32),
            pltpu.VMEM((CHUNK, PDIM), jnp.float32),
            pltpu.VMEM((CHUNK, PDIM), jnp.float32),
            pltpu.VMEM((BPW,), jnp.float32),
            pltpu.SemaphoreType.DMA,
            pltpu.SemaphoreType.DMA,
        ],
    )(_dot_body)
    return dot(scrh, scrab)


def kernel(prev, pos, neg, item_in_weight, item_out_weight):
    return _fmc(prev.astype(jnp.int32), pos.astype(jnp.int32),
                neg.astype(jnp.int32), item_in_weight, item_out_weight)


# speedup vs baseline: 2.9966x; 1.1056x over previous
import functools

import jax
import jax.numpy as jnp
from jax import lax
from jax.experimental import pallas as pl
from jax.experimental.pallas import tpu as pltpu
from jax.experimental.pallas import tpu_sc as plsc

DIM = 64
PDIM = 128
BATCH = 16384
NUM_CORES = 2
NUM_SUBCORES = 16
LANES = 16
NW = NUM_CORES * NUM_SUBCORES
BPW = BATCH // NW
NITEMS = 1000001
BLKW = 128
BPWORKER = 245
RNG = BPWORKER * BLKW
NFULL = (NITEMS // BLKW)
STAG = 32
NBUF = 3
WINW = 2
NWIN = (BPWORKER + WINW - 1) // WINW
CAP = 8192
CHUNK = 128
NCHUNK = BPW // CHUNK
GPC = CHUNK // LANES


def _gather_pass(table, tail, tailbuf, scratch, vals, mlist, sortl,
                 win0, win1, win2, stag, sidx, counts, offs, offs2,
                 semw0, semw1, semw2, nstream, dump_base, lo, hi,
                 blk0, nblk, is_last):
    n = nstream * BATCH
    nvreg = n // LANES
    iota = lax.iota(jnp.int32, LANES)
    wins = (win0, win1, win2)
    semws = (semw0, semw1, semw2)

    def wstart(w):
        return jnp.minimum((blk0 + WINW * w) * BLKW,
                           (NFULL - WINW) * BLKW)

    def fire(w, par):
        cp = pltpu.make_async_copy(
            table.at[:, pl.ds(wstart(w), WINW * BLKW)], wins[par], semws[par])
        cp.start()

    def wait_win(par):
        pltpu.make_async_copy(
            table.at[:, pl.ds(0, WINW * BLKW)], wins[par], semws[par]).wait()

    vals[pl.ds(n, LANES)] = jnp.full((LANES,), 2 * NITEMS, jnp.int32)

    def fill_dumps():
        for q in range(STAG // LANES):
            sidx[pl.ds(q * LANES, LANES)] = dump_base + q * LANES + iota

    fill_dumps()

    def extract_entry(e, r, win, base):
        pe = sortl[pl.ds(e, LANES)][0]
        vv = vals[pl.ds(pe, LANES)][0]
        lane = jnp.full((LANES,), vv - base, jnp.int32)
        for q in range(DIM // LANES):
            g = plsc.load_gather(win, [q * LANES + iota, lane])
            stag[r, pl.ds(q * LANES, LANES)] = g
        plsc.store_scatter(sidx, [jnp.full((LANES,), r, jnp.int32)],
                           jnp.full((LANES,), pe, jnp.int32),
                           mask=iota == 0)
        return r + 1

    def flush():
        pltpu.sync_copy(stag, scratch.at[sidx])
        fill_dumps()

    def entry_loop(e0, e1, r, win, base):
        def entry_step(e, rr):
            rr2 = extract_entry(e, rr, win, base)

            def do_flush(_):
                flush()
                return jnp.int32(0)

            return lax.cond(rr2 == STAG, do_flush, lambda _: rr2, 0)

        return lax.fori_loop(e0, e1, entry_step, r)

    def round_body(state):
        vstart, _ = state

        fire(0, 0)
        fire(1, 1)

        def scan_cond(s):
            i, cnt = s
            return jnp.logical_and(i < nvreg, cnt <= CAP - LANES)

        def scan_step(s):
            i, cnt = s
            v = vals[pl.ds(i * LANES, LANES)]
            posv = i * LANES + iota
            m = (v >= lo) & (v < hi)
            plsc.store_compressed(mlist.at[pl.ds(cnt, LANES)], posv, mask=m)
            return i + 1, cnt + plsc.all_reduce_population_count(m)[0]

        vend, cnt = lax.while_loop(scan_cond, scan_step,
                                   (vstart, jnp.int32(0)))
        mlist[pl.ds(cnt, LANES)] = jnp.full((LANES,), n, jnp.int32)
        cnt16 = (cnt + LANES - 1) // LANES

        def zero_step(b, _):
            counts[b] = 0
            return 0

        lax.fori_loop(0, 256, zero_step, 0)

        def count_step(i, _):
            mvec = mlist[pl.ds(i * LANES, LANES)]
            vv = plsc.load_gather(vals, [mvec])
            bv = jnp.minimum((vv - lo) >> 7, 255)
            for j in range(LANES):
                counts[bv[j]] = counts[bv[j]] + 1
            return 0

        lax.fori_loop(0, cnt16, count_step, 0)

        offs[0] = 0
        offs2[0] = 0

        def prefix(b, _):
            t = offs[b] + counts[b]
            offs[b + 1] = t
            offs2[b + 1] = t
            return 0

        lax.fori_loop(0, 256, prefix, 0)

        def place_step(i, _):
            mvec = mlist[pl.ds(i * LANES, LANES)]
            vv = plsc.load_gather(vals, [mvec])
            bv = jnp.minimum((vv - lo) >> 7, 255)
            for j in range(LANES):
                b = bv[j]
                o = offs[b]
                plsc.store_scatter(sortl, [jnp.full((LANES,), o, jnp.int32)],
                                   jnp.full((LANES,), mvec[j], jnp.int32),
                                   mask=iota == 0)
                offs[b] = o + 1
            return 0

        lax.fori_loop(0, cnt16, place_step, 0)

        def proc_window(w, r, win):
            e0 = offs2[WINW * w]
            e1 = offs2[jnp.minimum(WINW * (w + 1), nblk)]
            return entry_loop(e0, e1, r, win, wstart(w))

        def tri_step(i, r):
            for k in range(NBUF):
                w = NBUF * i + k
                wait_win(k)
                fire(w + 2, (k + 2) % NBUF)
                r = lax.cond(WINW * w < nblk,
                             functools.partial(proc_window, w, win=wins[k]),
                             lambda rr: rr, r)
            return r

        r = lax.fori_loop(0, NWIN // NBUF, tri_step, jnp.int32(0))
        wait_win(0)
        wait_win(1)

        def do_tail(rr):
            pltpu.sync_copy(tail, tailbuf)
            return entry_loop(offs2[nblk], offs2[nblk + 1], rr, tailbuf,
                              NFULL * BLKW)

        r = lax.cond(is_last, do_tail, lambda rr: rr, r)

        @pl.when(r > 0)
        def _final():
            flush()

        return vend, jnp.int32(0)

    lax.while_loop(lambda s: s[0] < nvreg, round_body, (jnp.int32(0),
                                                        jnp.int32(0)))


def _scan_body(prev_hbm, pos_hbm, neg_hbm, wint_hbm, woutt_hbm,
               tin_hbm, tout_hbm, scrh_hbm, scrab_hbm,
               vals, mlist, sortl, win0, win1, win2, stag, sidx, tailbuf,
               counts, offs, offs2, semw0, semw1, semw2):
    wid = lax.axis_index("s") * NUM_CORES + lax.axis_index("c")
    lo = wid * RNG
    hi = jnp.minimum(lo + RNG, NITEMS)
    blk0 = wid * BPWORKER
    nblk = jnp.minimum(BPWORKER, NFULL - blk0)
    is_last = wid == NW - 1

    pltpu.sync_copy(prev_hbm, vals.at[pl.ds(0, BATCH)])
    _gather_pass(wint_hbm, tin_hbm, tailbuf, scrh_hbm, vals, mlist, sortl,
                 win0, win1, win2, stag, sidx, counts, offs, offs2,
                 semw0, semw1, semw2, 1, BATCH, lo, hi, blk0, nblk, is_last)

    pltpu.sync_copy(pos_hbm, vals.at[pl.ds(0, BATCH)])
    pltpu.sync_copy(neg_hbm, vals.at[pl.ds(BATCH, BATCH)])
    _gather_pass(woutt_hbm, tout_hbm, tailbuf, scrab_hbm, vals, mlist, sortl,
                 win0, win1, win2, stag, sidx, counts, offs, offs2,
                 semw0, semw1, semw2, 2, 2 * BATCH, lo, hi, blk0, nblk,
                 is_last)


def _dot_body(scrh_hbm, scrab_hbm, out_hbm,
              h0, h1, p0, p1, n0, n1, outv, sem0, sem1):
    wid = lax.axis_index("s") * NUM_CORES + lax.axis_index("c")
    base = wid * BPW
    hbufs = (h0, h1)
    pbufs = (p0, p1)
    nbufs = (n0, n1)
    sems = (sem0, sem1)

    def fire(c):
        sl = pl.ds(base + c * CHUNK, CHUNK)
        sl2 = pl.ds(BATCH + base + c * CHUNK, CHUNK)
        sem = sems[c % 2]
        cps = (
            pltpu.make_async_copy(scrh_hbm.at[sl], hbufs[c % 2], sem),
            pltpu.make_async_copy(scrab_hbm.at[sl], pbufs[c % 2], sem),
            pltpu.make_async_copy(scrab_hbm.at[sl2], nbufs[c % 2], sem),
        )
        for cp in cps:
            cp.start()
        return cps

    iota = lax.iota(jnp.int32, LANES)
    inflight = fire(0)
    for c in range(NCHUNK):
        for cp in inflight:
            cp.wait()
        if c + 1 < NCHUNK:
            inflight = fire(c + 1)
        hb, pb, nb = hbufs[c % 2], pbufs[c % 2], nbufs[c % 2]

        def group(g, _):
            rows = g * LANES + iota
            acc = jnp.zeros((LANES,), jnp.float32)
            for d in range(DIM):
                col = jnp.full((LANES,), d, jnp.int32)
                hv = plsc.load_gather(hb, [rows, col])
                pv = plsc.load_gather(pb, [rows, col])
                nv = plsc.load_gather(nb, [rows, col])
                acc = acc + hv * (pv - nv)
            outv[pl.ds(c * CHUNK + g * LANES, LANES)] = acc
            return 0

        lax.fori_loop(0, GPC, group, 0)

    pltpu.sync_copy(outv, out_hbm.at[pl.ds(base, BPW)])


@jax.jit
def _fmc(prev, pos, neg, item_in_weight, item_out_weight):
    wint = item_in_weight.T
    woutt = item_out_weight.T
    pad = ((0, 0), (0, BLKW - (NITEMS - NFULL * BLKW)))
    tin = jnp.pad(item_in_weight[NFULL * BLKW:].T, pad)
    tout = jnp.pad(item_out_weight[NFULL * BLKW:].T, pad)
    mesh = plsc.VectorSubcoreMesh(core_axis_name="c", subcore_axis_name="s")

    scan = functools.partial(
        pl.kernel,
        out_type=(
            jax.ShapeDtypeStruct((BATCH + STAG, PDIM), jnp.float32),
            jax.ShapeDtypeStruct((2 * BATCH + STAG, PDIM), jnp.float32),
        ),
        mesh=mesh,
        compiler_params=pltpu.CompilerParams(needs_layout_passes=False),
        scratch_types=[
            pltpu.VMEM((2 * BATCH + LANES,), jnp.int32),
            pltpu.VMEM((CAP + 2 * LANES,), jnp.int32),
            pltpu.VMEM((CAP + 2 * LANES,), jnp.int32),
            pltpu.VMEM((DIM, WINW * BLKW), jnp.float32),
            pltpu.VMEM((DIM, WINW * BLKW), jnp.float32),
            pltpu.VMEM((DIM, WINW * BLKW), jnp.float32),
            pltpu.VMEM((STAG, PDIM), jnp.float32),
            pltpu.VMEM((STAG,), jnp.int32),
            pltpu.VMEM((DIM, BLKW), jnp.float32),
            pltpu.SMEM((258,), jnp.int32),
            pltpu.SMEM((258,), jnp.int32),
            pltpu.SMEM((258,), jnp.int32),
            pltpu.SemaphoreType.DMA,
            pltpu.SemaphoreType.DMA,
            pltpu.SemaphoreType.DMA,
        ],
    )(_scan_body)
    scrh, scrab = scan(prev, pos, neg, wint, woutt, tin, tout)

    dot = functools.partial(
        pl.kernel,
        out_type=jax.ShapeDtypeStruct((BATCH,), jnp.float32),
        mesh=mesh,
        compiler_params=pltpu.CompilerParams(needs_layout_passes=False),
        scratch_types=[
            pltpu.VMEM((CHUNK, PDIM), jnp.float32),
            pltpu.VMEM((CHUNK, PDIM), jnp.float32),
            pltpu.VMEM((CHUNK, PDIM), jnp.float32),
            pltpu.VMEM((CHUNK, PDIM), jnp.float32),
            pltpu.VMEM((CHUNK, PDIM), jnp.float32),
            pltpu.VMEM((CHUNK, PDIM), jnp.float32),
            pltpu.VMEM((BPW,), jnp.float32),
            pltpu.SemaphoreType.DMA,
            pltpu.SemaphoreType.DMA,
        ],
    )(_dot_body)
    return dot(scrh, scrab)


def kernel(prev, pos, neg, item_in_weight, item_out_weight):
    return _fmc(prev.astype(jnp.int32), pos.astype(jnp.int32),
                neg.astype(jnp.int32), item_in_weight, item_out_weight)


# speedup vs baseline: 3.1063x; 1.0366x over previous
import functools

import jax
import jax.numpy as jnp
from jax import lax
from jax.experimental import pallas as pl
from jax.experimental.pallas import tpu as pltpu
from jax.experimental.pallas import tpu_sc as plsc

DIM = 64
PDIM = 128
BATCH = 16384
NUM_CORES = 2
NUM_SUBCORES = 16
LANES = 16
NW = NUM_CORES * NUM_SUBCORES
BPW = BATCH // NW
NITEMS = 1000001
BLKW = 128
BPWORKER = 245
RNG = BPWORKER * BLKW
NFULL = (NITEMS // BLKW)
STAG = 32
NBUF = 3
WINW = 3
NWIN = (BPWORKER + WINW - 1) // WINW
CAP = 8192
CHUNK = 128
NCHUNK = BPW // CHUNK
GPC = CHUNK // LANES


def _gather_pass(table, tail, scratch, vals, mlist, sortl,
                 win0, win1, win2, stag, sidx, counts, offs, offs2,
                 semw0, semw1, semw2, nstream, dump_base, lo, hi,
                 blk0, nblk, is_last):
    n = nstream * BATCH
    nvreg = n // LANES
    iota = lax.iota(jnp.int32, LANES)
    wins = (win0, win1, win2)
    semws = (semw0, semw1, semw2)

    def wstart(w):
        return jnp.minimum((blk0 + WINW * w) * BLKW,
                           (NFULL - WINW) * BLKW)

    def fire(w, par):
        cp = pltpu.make_async_copy(
            table.at[:, pl.ds(wstart(w), WINW * BLKW)], wins[par], semws[par])
        cp.start()

    def wait_win(par):
        pltpu.make_async_copy(
            table.at[:, pl.ds(0, WINW * BLKW)], wins[par], semws[par]).wait()

    vals[pl.ds(n, LANES)] = jnp.full((LANES,), 2 * NITEMS, jnp.int32)

    def fill_dumps():
        for q in range(STAG // LANES):
            sidx[pl.ds(q * LANES, LANES)] = dump_base + q * LANES + iota

    fill_dumps()

    def extract_entry(e, r, win, base):
        pe = sortl[pl.ds(e, LANES)][0]
        vv = vals[pl.ds(pe, LANES)][0]
        lane = jnp.full((LANES,), vv - base, jnp.int32)
        for q in range(DIM // LANES):
            g = plsc.load_gather(win, [q * LANES + iota, lane])
            stag[r, pl.ds(q * LANES, LANES)] = g
        plsc.store_scatter(sidx, [jnp.full((LANES,), r, jnp.int32)],
                           jnp.full((LANES,), pe, jnp.int32),
                           mask=iota == 0)
        return r + 1

    def flush():
        pltpu.sync_copy(stag, scratch.at[sidx])
        fill_dumps()

    def entry_loop(e0, e1, r, win, base):
        def entry_step(e, rr):
            rr2 = extract_entry(e, rr, win, base)

            def do_flush(_):
                flush()
                return jnp.int32(0)

            return lax.cond(rr2 == STAG, do_flush, lambda _: rr2, 0)

        return lax.fori_loop(e0, e1, entry_step, r)

    def round_body(state):
        vstart, _ = state

        fire(0, 0)
        fire(1, 1)

        def scan_cond(s):
            i, cnt = s
            return jnp.logical_and(i < nvreg, cnt <= CAP - LANES)

        def scan_step(s):
            i, cnt = s
            v = vals[pl.ds(i * LANES, LANES)]
            posv = i * LANES + iota
            m = (v >= lo) & (v < hi)
            plsc.store_compressed(mlist.at[pl.ds(cnt, LANES)], posv, mask=m)
            return i + 1, cnt + plsc.all_reduce_population_count(m)[0]

        vend, cnt = lax.while_loop(scan_cond, scan_step,
                                   (vstart, jnp.int32(0)))
        mlist[pl.ds(cnt, LANES)] = jnp.full((LANES,), n, jnp.int32)
        cnt16 = (cnt + LANES - 1) // LANES

        def zero_step(b, _):
            counts[b] = 0
            return 0

        lax.fori_loop(0, 256, zero_step, 0)

        def count_step(i, _):
            mvec = mlist[pl.ds(i * LANES, LANES)]
            vv = plsc.load_gather(vals, [mvec])
            bv = jnp.minimum((vv - lo) >> 7, 255)
            for j in range(LANES):
                counts[bv[j]] = counts[bv[j]] + 1
            return 0

        lax.fori_loop(0, cnt16, count_step, 0)

        offs[0] = 0
        offs2[0] = 0

        def prefix(b, _):
            t = offs[b] + counts[b]
            offs[b + 1] = t
            offs2[b + 1] = t
            return 0

        lax.fori_loop(0, 256, prefix, 0)

        def place_step(i, _):
            mvec = mlist[pl.ds(i * LANES, LANES)]
            vv = plsc.load_gather(vals, [mvec])
            bv = jnp.minimum((vv - lo) >> 7, 255)
            for j in range(LANES):
                b = bv[j]
                o = offs[b]
                plsc.store_scatter(sortl, [jnp.full((LANES,), o, jnp.int32)],
                                   jnp.full((LANES,), mvec[j], jnp.int32),
                                   mask=iota == 0)
                offs[b] = o + 1
            return 0

        lax.fori_loop(0, cnt16, place_step, 0)

        def proc_window(w, r, win):
            e0 = offs2[WINW * w]
            e1 = offs2[jnp.minimum(WINW * (w + 1), nblk)]
            return entry_loop(e0, e1, r, win, wstart(w))

        def tri_step(i, r):
            for k in range(NBUF):
                w = NBUF * i + k
                wait_win(k)
                fire(w + 2, (k + 2) % NBUF)
                r = lax.cond(WINW * w < nblk,
                             functools.partial(proc_window, w, win=wins[k]),
                             lambda rr: rr, r)
            return r

        r = lax.fori_loop(0, (NWIN + NBUF - 1) // NBUF, tri_step,
                          jnp.int32(0))
        wait_win(0)
        wait_win(1)

        def do_tail(rr):
            pltpu.sync_copy(tail, win0.at[:, pl.ds(0, BLKW)])
            return entry_loop(offs2[nblk], offs2[nblk + 1], rr, win0,
                              NFULL * BLKW)

        r = lax.cond(is_last, do_tail, lambda rr: rr, r)

        @pl.when(r > 0)
        def _final():
            flush()

        return vend, jnp.int32(0)

    lax.while_loop(lambda s: s[0] < nvreg, round_body, (jnp.int32(0),
                                                        jnp.int32(0)))


def _scan_body(prev_hbm, pos_hbm, neg_hbm, wint_hbm, woutt_hbm,
               tin_hbm, tout_hbm, scrh_hbm, scrab_hbm,
               vals, mlist, sortl, win0, win1, win2, stag, sidx,
               counts, offs, offs2, semw0, semw1, semw2):
    wid = lax.axis_index("s") * NUM_CORES + lax.axis_index("c")
    lo = wid * RNG
    hi = jnp.minimum(lo + RNG, NITEMS)
    blk0 = wid * BPWORKER
    nblk = jnp.minimum(BPWORKER, NFULL - blk0)
    is_last = wid == NW - 1

    pltpu.sync_copy(prev_hbm, vals.at[pl.ds(0, BATCH)])
    _gather_pass(wint_hbm, tin_hbm, scrh_hbm, vals, mlist, sortl,
                 win0, win1, win2, stag, sidx, counts, offs, offs2,
                 semw0, semw1, semw2, 1, BATCH, lo, hi, blk0, nblk, is_last)

    pltpu.sync_copy(pos_hbm, vals.at[pl.ds(0, BATCH)])
    pltpu.sync_copy(neg_hbm, vals.at[pl.ds(BATCH, BATCH)])
    _gather_pass(woutt_hbm, tout_hbm, scrab_hbm, vals, mlist, sortl,
                 win0, win1, win2, stag, sidx, counts, offs, offs2,
                 semw0, semw1, semw2, 2, 2 * BATCH, lo, hi, blk0, nblk,
                 is_last)


def _dot_body(scrh_hbm, scrab_hbm, out_hbm,
              h0, h1, p0, p1, n0, n1, outv, sem0, sem1):
    wid = lax.axis_index("s") * NUM_CORES + lax.axis_index("c")
    base = wid * BPW
    hbufs = (h0, h1)
    pbufs = (p0, p1)
    nbufs = (n0, n1)
    sems = (sem0, sem1)

    def fire(c):
        sl = pl.ds(base + c * CHUNK, CHUNK)
        sl2 = pl.ds(BATCH + base + c * CHUNK, CHUNK)
        sem = sems[c % 2]
        cps = (
            pltpu.make_async_copy(scrh_hbm.at[sl], hbufs[c % 2], sem),
            pltpu.make_async_copy(scrab_hbm.at[sl], pbufs[c % 2], sem),
            pltpu.make_async_copy(scrab_hbm.at[sl2], nbufs[c % 2], sem),
        )
        for cp in cps:
            cp.start()
        return cps

    iota = lax.iota(jnp.int32, LANES)
    inflight = fire(0)
    for c in range(NCHUNK):
        for cp in inflight:
            cp.wait()
        if c + 1 < NCHUNK:
            inflight = fire(c + 1)
        hb, pb, nb = hbufs[c % 2], pbufs[c % 2], nbufs[c % 2]

        def group(g, _):
            rows = g * LANES + iota
            acc = jnp.zeros((LANES,), jnp.float32)
            for d in range(DIM):
                col = jnp.full((LANES,), d, jnp.int32)
                hv = plsc.load_gather(hb, [rows, col])
                pv = plsc.load_gather(pb, [rows, col])
                nv = plsc.load_gather(nb, [rows, col])
                acc = acc + hv * (pv - nv)
            outv[pl.ds(c * CHUNK + g * LANES, LANES)] = acc
            return 0

        lax.fori_loop(0, GPC, group, 0)

    pltpu.sync_copy(outv, out_hbm.at[pl.ds(base, BPW)])


@jax.jit
def _fmc(prev, pos, neg, item_in_weight, item_out_weight):
    wint = item_in_weight.T
    woutt = item_out_weight.T
    pad = ((0, 0), (0, BLKW - (NITEMS - NFULL * BLKW)))
    tin = jnp.pad(item_in_weight[NFULL * BLKW:].T, pad)
    tout = jnp.pad(item_out_weight[NFULL * BLKW:].T, pad)
    mesh = plsc.VectorSubcoreMesh(core_axis_name="c", subcore_axis_name="s")

    scan = functools.partial(
        pl.kernel,
        out_type=(
            jax.ShapeDtypeStruct((BATCH + STAG, PDIM), jnp.float32),
            jax.ShapeDtypeStruct((2 * BATCH + STAG, PDIM), jnp.float32),
        ),
        mesh=mesh,
        compiler_params=pltpu.CompilerParams(needs_layout_passes=False),
        scratch_types=[
            pltpu.VMEM((2 * BATCH + LANES,), jnp.int32),
            pltpu.VMEM((CAP + 2 * LANES,), jnp.int32),
            pltpu.VMEM((CAP + 2 * LANES,), jnp.int32),
            pltpu.VMEM((DIM, WINW * BLKW), jnp.float32),
            pltpu.VMEM((DIM, WINW * BLKW), jnp.float32),
            pltpu.VMEM((DIM, WINW * BLKW), jnp.float32),
            pltpu.VMEM((STAG, PDIM), jnp.float32),
            pltpu.VMEM((STAG,), jnp.int32),
            pltpu.SMEM((258,), jnp.int32),
            pltpu.SMEM((258,), jnp.int32),
            pltpu.SMEM((258,), jnp.int32),
            pltpu.SemaphoreType.DMA,
            pltpu.SemaphoreType.DMA,
            pltpu.SemaphoreType.DMA,
        ],
    )(_scan_body)
    scrh, scrab = scan(prev, pos, neg, wint, woutt, tin, tout)

    dot = functools.partial(
        pl.kernel,
        out_type=jax.ShapeDtypeStruct((BATCH,), jnp.float32),
        mesh=mesh,
        compiler_params=pltpu.CompilerParams(needs_layout_passes=False),
        scratch_types=[
            pltpu.VMEM((CHUNK, PDIM), jnp.float32),
            pltpu.VMEM((CHUNK, PDIM), jnp.float32),
            pltpu.VMEM((CHUNK, PDIM), jnp.float32),
            pltpu.VMEM((CHUNK, PDIM), jnp.float32),
            pltpu.VMEM((CHUNK, PDIM), jnp.float32),
            pltpu.VMEM((CHUNK, PDIM), jnp.float32),
            pltpu.VMEM((BPW,), jnp.float32),
            pltpu.SemaphoreType.DMA,
            pltpu.SemaphoreType.DMA,
        ],
    )(_dot_body)
    return dot(scrh, scrab)


def kernel(prev, pos, neg, item_in_weight, item_out_weight):
    return _fmc(prev.astype(jnp.int32), pos.astype(jnp.int32),
                neg.astype(jnp.int32), item_in_weight, item_out_weight)
